# baseline (speedup 1.0000x reference)
import functools

import jax
import jax.numpy as jnp
from jax import lax
from jax.experimental import pallas as pl
from jax.experimental.pallas import tpu as pltpu
from jax.experimental.pallas import tpu_sc as plsc

N_NODES = 50000
SEQ_LEN = 8
N_FEAT = 128
HIDDEN = 32
N_EDGES = 624000
N_FLAT = N_NODES * SEQ_LEN

NC = 2
NS = 16

EROWS = N_EDGES // 128
CHUNK_ROWS = 4
N_FULL_CHUNKS = EROWS // CHUNK_ROWS
REM_ROWS = EROWS - N_FULL_CHUNKS * CHUNK_ROWS
HIST_REM_W = N_FULL_CHUNKS % (2 * 16)
EDGE_REM_S = N_FULL_CHUNKS % 16

N_PASSES = 4
RANGE = N_FLAT // (NC * N_PASSES)
DUMP = RANGE
CP_ROWS = 80
N_CP = RANGE // CP_ROWS
HIST_CP = 4000
HIST_NCP = N_FLAT // HIST_CP

@functools.lru_cache(maxsize=1)
def _get_mesh():
    return plsc.VectorSubcoreMesh(
        core_axis_name="c", subcore_axis_name="s",
        num_cores=NC, num_subcores=NS)


def _hist_body(dst_hbm, deg0_hbm, deg1_hbm, acc, dstv, ones_v, cbuf):
    c = lax.axis_index("c")
    s = lax.axis_index("s")
    w = c * NS + s

    zeros16 = jnp.zeros((16,), jnp.float32)
    ones16 = jnp.ones((16,), jnp.float32)

    def _fill(i, _):
        ones_v[pl.ds(i * 16, 16)] = ones16
        return 0
    lax.fori_loop(0, 8, _fill, 0)

    def _zcb(i, _):
        cbuf[pl.ds(i * 16, 16)] = zeros16
        return 0
    lax.fori_loop(0, HIST_CP // 16, _zcb, 0)

    def _zacc(i, _):
        m = s + i * NS

        @pl.when(m < HIST_NCP)
        def _():
            pltpu.sync_copy(cbuf, acc.at[pl.ds(m * HIST_CP, HIST_CP)])
        return 0
    lax.fori_loop(0, (HIST_NCP + NS - 1) // NS, _zacc, 0)
    plsc.subcore_barrier()

    def _scatter_rows(nrows):
        def go(_=None):
            for q in range(nrows):
                pltpu.sync_copy(ones_v, acc.at[dstv.at[q]], add=True)
        return go

    def _chunk(i, _):
        k = w + i * (NC * NS)

        @pl.when(k < N_FULL_CHUNKS)
        def _():
            pltpu.sync_copy(dst_hbm.at[pl.ds(k * CHUNK_ROWS, CHUNK_ROWS)], dstv)
            _scatter_rows(CHUNK_ROWS)()
        return 0
    lax.fori_loop(0, (N_FULL_CHUNKS + NC * NS - 1) // (NC * NS), _chunk, 0)

    @pl.when(w == HIST_REM_W)
    def _():
        pltpu.sync_copy(
            dst_hbm.at[pl.ds(N_FULL_CHUNKS * CHUNK_ROWS, REM_ROWS)],
            dstv.at[pl.ds(0, REM_ROWS)])
        _scatter_rows(REM_ROWS)()

    plsc.subcore_barrier()

    def _copy_out(i, _):
        m = s + i * NS

        @pl.when(m < HIST_NCP)
        def _():
            off = m * HIST_CP
            pltpu.sync_copy(acc.at[pl.ds(off, HIST_CP)], cbuf)

            @pl.when(c == 0)
            def _():
                pltpu.sync_copy(cbuf, deg0_hbm.at[pl.ds(off, HIST_CP)])

            @pl.when(c == 1)
            def _():
                pltpu.sync_copy(cbuf, deg1_hbm.at[pl.ds(off, HIST_CP)])
        return 0
    lax.fori_loop(0, (HIST_NCP + NS - 1) // NS, _copy_out, 0)


@functools.lru_cache(maxsize=1)
def _hist_kernel():
    return pl.kernel(
        _hist_body,
        out_type=(jax.ShapeDtypeStruct((N_FLAT,), jnp.float32),
                  jax.ShapeDtypeStruct((N_FLAT,), jnp.float32)),
        mesh=_get_mesh(),
        scratch_types=dict(
            acc=pltpu.VMEM_SHARED((N_FLAT,), jnp.float32),
            dstv=pltpu.VMEM((CHUNK_ROWS, 128), jnp.int32),
            ones_v=pltpu.VMEM((128,), jnp.float32),
            cbuf=pltpu.VMEM((HIST_CP,), jnp.float32),
        ),
        compiler_params=pltpu.CompilerParams(use_tc_tiling_on_sc=False),
    )


def _mm_body(x_ref, w_ref, deg0_ref, deg1_ref, hh_ref, dinv_ref):
    dsum = deg0_ref[...] + deg1_ref[...] + 1.0
    dinv = lax.rsqrt(dsum)
    h = jnp.dot(x_ref[...], w_ref[...], preferred_element_type=jnp.float32)
    hh_ref[...] = h * dinv
    dinv_ref[...] = dinv


def _run_mm(x_in, gcn_W, deg0, deg1):
    R = 3200
    grid = (N_FLAT // R,)
    return pl.pallas_call(
        _mm_body,
        grid=grid,
        in_specs=[
            pl.BlockSpec((R, N_FEAT), lambda i: (i, 0)),
            pl.BlockSpec((N_FEAT, HIDDEN), lambda i: (0, 0)),
            pl.BlockSpec((R, 1), lambda i: (i, 0)),
            pl.BlockSpec((R, 1), lambda i: (i, 0)),
        ],
        out_specs=[
            pl.BlockSpec((R, HIDDEN), lambda i: (i, 0)),
            pl.BlockSpec((R, 1), lambda i: (i, 0)),
        ],
        out_shape=[
            jax.ShapeDtypeStruct((N_FLAT, HIDDEN), jnp.float32),
            jax.ShapeDtypeStruct((N_FLAT, 1), jnp.float32),
        ],
    )(x_in, gcn_W, deg0.reshape(N_FLAT, 1), deg1.reshape(N_FLAT, 1))


def _edge_body(src_hbm, dst_hbm, hh_hbm, seg_hbm,
               acc, srcv, dstv, sel_src, sel_rel, rel_stage, rows,
               cbuf, semL, semG, semS):
    c = lax.axis_index("c")
    s = lax.axis_index("s")

    zeros16i = jnp.zeros((16,), jnp.int32)
    dump16 = jnp.full((16,), DUMP, jnp.int32)
    lane16 = jax.lax.broadcasted_iota(jnp.int32, (16,), 0)

    def _issue_loads(k):
        off = k * CHUNK_ROWS
        pltpu.async_copy(src_hbm.at[pl.ds(off, CHUNK_ROWS)], srcv, semL)
        pltpu.async_copy(dst_hbm.at[pl.ds(off, CHUNK_ROWS)], dstv, semL)

    def _wait_loads():
        pltpu.make_async_copy(src_hbm.at[pl.ds(0, CHUNK_ROWS)], srcv, semL).wait()
        pltpu.make_async_copy(dst_hbm.at[pl.ds(0, CHUNK_ROWS)], dstv, semL).wait()

    def _compress(nrows, base, cnt):
        def _cv(jj, cnt):
            for u in range(4):
                j = jj * 4 + u
                q = j // 8
                col = (j % 8) * 16
                d = dstv[q, pl.ds(col, 16)]
                r = d - base
                m = (r >= 0) & (r < RANGE)
                mi = jnp.where(m, 1, 0).astype(jnp.int32)
                pf = plsc.cumsum(mi)
                pos = jnp.where(m, cnt + pf - 1, 1535)
                plsc.store_scatter(sel_src, [pos], srcv[q, pl.ds(col, 16)])
                plsc.store_scatter(sel_rel, [pos], jnp.where(m, r, DUMP))
                cnt = cnt + jnp.sum(mi)
            return cnt
        cnt = lax.fori_loop(0, nrows * 2, _cv, cnt)
        pad = (-cnt) & 7
        sel_src[pl.ds(cnt, 16)] = zeros16i
        sel_rel[pl.ds(cnt, 16)] = dump16
        return cnt + pad

    def _flush512(off0):
        off0 = pl.multiple_of(off0, 8)
        gds = [pltpu.async_copy(
                   hh_hbm.at[sel_src.at[pl.ds(off0 + q * 128, 128)]],
                   rows.at[pl.ds(q * 128, 128)], semG)
               for q in range(4)]
        def _mv(j, _):
            q = j // 8
            col = (j % 8) * 16
            rel_stage[q, pl.ds(col, 16)] = sel_rel[pl.ds(off0 + q * 128 + col, 16)]
            return 0
        lax.fori_loop(0, 32, _mv, 0)
        for gd in gds:
            gd.wait()
        sds = [pltpu.async_copy(rows.at[pl.ds(q * 128, 128)],
                                acc.at[rel_stage.at[q]], semS, add=True)
               for q in range(4)]
        for sd in sds:
            sd.wait()

    def _maybe_flush(cnt):
        def fl(cnt):
            _flush512(cnt - 512)
            return cnt - 512
        return lax.cond(cnt >= 512, fl, lambda cnt: cnt, cnt)

    for p in range(N_PASSES):
        base = (c * N_PASSES + p) * RANGE

        def _iacc(i, _):
            m = s + i * NS

            @pl.when(m < N_CP)
            def _():
                pltpu.sync_copy(hh_hbm.at[pl.ds(base + m * CP_ROWS, CP_ROWS)],
                                acc.at[pl.ds(m * CP_ROWS, CP_ROWS)])
            return 0
        lax.fori_loop(0, (N_CP + NS - 1) // NS, _iacc, 0)
        plsc.subcore_barrier()

        _issue_loads(s)

        def _chunk(i, cnt):
            k = s + i * NS

            def do(cnt):
                _wait_loads()
                cnt = _compress(CHUNK_ROWS, base, cnt)

                @pl.when(k + NS < N_FULL_CHUNKS)
                def _():
                    _issue_loads(k + NS)
                cnt = _maybe_flush(cnt)
                cnt = _maybe_flush(cnt)
                return cnt
            return lax.cond(k < N_FULL_CHUNKS, do, lambda cnt: cnt, cnt)
        cnt = lax.fori_loop(0, (N_FULL_CHUNKS + NS - 1) // NS, _chunk, 0)

        def _rem(cnt):
            pltpu.sync_copy(src_hbm.at[pl.ds(N_FULL_CHUNKS * CHUNK_ROWS, REM_ROWS)],
                            srcv.at[pl.ds(0, REM_ROWS)])
            pltpu.sync_copy(dst_hbm.at[pl.ds(N_FULL_CHUNKS * CHUNK_ROWS, REM_ROWS)],
                            dstv.at[pl.ds(0, REM_ROWS)])
            cnt = _compress(REM_ROWS, base, cnt)
            return _maybe_flush(cnt)
        cnt = lax.cond(s == EDGE_REM_S, _rem, lambda cnt: cnt, cnt)

        def _fin(cnt):
            def _padloop(i, _):
                off = cnt + i * 16

                @pl.when(off < 512)
                def _():
                    sel_src[pl.ds(off, 16)] = zeros16i
                    sel_rel[pl.ds(off, 16)] = dump16
                return 0
            lax.fori_loop(0, 32, _padloop, 0)
            _flush512(0)
            return 0
        _ = lax.cond(cnt > 0, _fin, lambda cnt: 0, cnt)

        plsc.subcore_barrier()

        def _cacc(i, _):
            m = s + i * NS

            @pl.when(m < N_CP)
            def _():
                row = m * CP_ROWS
                pltpu.sync_copy(acc.at[pl.ds(row, CP_ROWS)], cbuf)
                pltpu.sync_copy(cbuf, seg_hbm.at[pl.ds(base + row, CP_ROWS)])
            return 0
        lax.fori_loop(0, (N_CP + NS - 1) // NS, _cacc, 0)


@functools.lru_cache(maxsize=1)
def _edge_kernel():
    return pl.kernel(
        _edge_body,
        out_type=jax.ShapeDtypeStruct((N_FLAT, HIDDEN), jnp.float32),
        mesh=_get_mesh(),
        scratch_types=dict(
        acc=pltpu.VMEM_SHARED((RANGE + 1, HIDDEN), jnp.float32),
        srcv=pltpu.VMEM((CHUNK_ROWS, 128), jnp.int32),
        dstv=pltpu.VMEM((CHUNK_ROWS, 128), jnp.int32),
        sel_src=pltpu.VMEM((1536,), jnp.int32),
        sel_rel=pltpu.VMEM((1536,), jnp.int32),
        rel_stage=pltpu.VMEM((4, 128), jnp.int32),
        rows=pltpu.VMEM((512, HIDDEN), jnp.float32),
            cbuf=pltpu.VMEM((CP_ROWS, HIDDEN), jnp.float32),
            semL=pltpu.SemaphoreType.DMA,
            semG=pltpu.SemaphoreType.DMA,
            semS=pltpu.SemaphoreType.DMA,
        ),
        compiler_params=pltpu.CompilerParams(
            use_tc_tiling_on_sc=False, needs_layout_passes=False),
    )


def _lstm_body(seg_ref, dinv_ref, gcn_b_ref, wcat_ref,
               b4_ref, fcw_ref, fcb_ref, out_ref):
    B = seg_ref.shape[0]
    gb = gcn_b_ref[...]
    wcat = wcat_ref[...]
    b4 = b4_ref[...]
    hs = jnp.zeros((B, HIDDEN), jnp.float32)
    cs = jnp.zeros((B, HIDDEN), jnp.float32)
    for t in range(SEQ_LEN):
        xt = jax.nn.relu(dinv_ref[:, t, :] * seg_ref[:, t, :] + gb)
        xh = jnp.concatenate([xt, hs], axis=1)
        gates = jnp.dot(xh, wcat, preferred_element_type=jnp.float32) + b4
        gi = jax.nn.sigmoid(gates[:, 0 * HIDDEN:1 * HIDDEN])
        gf = jax.nn.sigmoid(gates[:, 1 * HIDDEN:2 * HIDDEN])
        gg = jnp.tanh(gates[:, 2 * HIDDEN:3 * HIDDEN])
        go = jax.nn.sigmoid(gates[:, 3 * HIDDEN:4 * HIDDEN])
        cs = gf * cs + gi * gg
        hs = go * jnp.tanh(cs)
    out_ref[...] = (jnp.dot(hs, fcw_ref[...], preferred_element_type=jnp.float32)
                    + fcb_ref[...])


def _run_lstm(seg3, dinv3, gcn_b2, wcat, b4, fcwT, fcb2):
    B = 1000
    grid = (N_NODES // B,)
    return pl.pallas_call(
        _lstm_body,
        grid=grid,
        in_specs=[
            pl.BlockSpec((B, SEQ_LEN, HIDDEN), lambda i: (i, 0, 0)),
            pl.BlockSpec((B, SEQ_LEN, 1), lambda i: (i, 0, 0)),
            pl.BlockSpec((1, HIDDEN), lambda i: (0, 0)),
            pl.BlockSpec((2 * HIDDEN, 4 * HIDDEN), lambda i: (0, 0)),
            pl.BlockSpec((1, 4 * HIDDEN), lambda i: (0, 0)),
            pl.BlockSpec((HIDDEN, 1), lambda i: (0, 0)),
            pl.BlockSpec((1, 1), lambda i: (0, 0)),
        ],
        out_specs=pl.BlockSpec((B, 1), lambda i: (i, 0)),
        out_shape=jax.ShapeDtypeStruct((N_NODES, 1), jnp.float32),
    )(seg3, dinv3, gcn_b2, wcat, b4, fcwT, fcb2)


def kernel(x, edge_index, gcn_W, gcn_b, W_ih, W_hh, b_ih, b_hh, fc_W, fc_b):
    x_in = x.reshape(N_FLAT, N_FEAT)
    src2 = edge_index[0].reshape(EROWS, 128)
    dst2 = edge_index[1].reshape(EROWS, 128)

    deg0, deg1 = _hist_kernel()(dst2)
    hh, dinv = _run_mm(x_in, gcn_W, deg0, deg1)
    segsum = _edge_kernel()(src2, dst2, hh)

    out = _run_lstm(
        segsum.reshape(N_NODES, SEQ_LEN, HIDDEN),
        dinv.reshape(N_NODES, SEQ_LEN, 1),
        gcn_b.reshape(1, HIDDEN),
        jnp.concatenate([W_ih.T, W_hh.T], axis=0),
        (b_ih + b_hh).reshape(1, 4 * HIDDEN),
        fc_W.T,
        fc_b.reshape(1, 1),
    )
    return out


# speedup vs baseline: 1.0227x; 1.0227x over previous
import functools

import jax
import jax.numpy as jnp
from jax import lax
from jax.experimental import pallas as pl
from jax.experimental.pallas import tpu as pltpu
from jax.experimental.pallas import tpu_sc as plsc

N_NODES = 50000
SEQ_LEN = 8
N_FEAT = 128
HIDDEN = 32
N_EDGES = 624000
N_FLAT = N_NODES * SEQ_LEN

NC = 2
NS = 16

EROWS = N_EDGES // 128
CHUNK_ROWS = 4
N_FULL_CHUNKS = EROWS // CHUNK_ROWS
REM_ROWS = EROWS - N_FULL_CHUNKS * CHUNK_ROWS
HIST_REM_W = N_FULL_CHUNKS % (2 * 16)
EDGE_REM_S = N_FULL_CHUNKS % 16

N_PASSES = 4
RANGE = N_FLAT // (NC * N_PASSES)
DUMP = RANGE
CP_ROWS = 80
N_CP = RANGE // CP_ROWS
HIST_CP = 4000
HIST_NCP = N_FLAT // HIST_CP

@functools.lru_cache(maxsize=1)
def _get_mesh():
    return plsc.VectorSubcoreMesh(
        core_axis_name="c", subcore_axis_name="s",
        num_cores=NC, num_subcores=NS)


def _hist_body(dst_hbm, deg0_hbm, deg1_hbm, acc, dstv, ones_v, cbuf):
    c = lax.axis_index("c")
    s = lax.axis_index("s")
    w = c * NS + s

    zeros16 = jnp.zeros((16,), jnp.float32)
    ones16 = jnp.ones((16,), jnp.float32)

    def _fill(i, _):
        ones_v[pl.ds(i * 16, 16)] = ones16
        return 0
    lax.fori_loop(0, 8, _fill, 0)

    def _zcb(i, _):
        cbuf[pl.ds(i * 16, 16)] = zeros16
        return 0
    lax.fori_loop(0, HIST_CP // 16, _zcb, 0)

    def _zacc(i, _):
        m = s + i * NS

        @pl.when(m < HIST_NCP)
        def _():
            pltpu.sync_copy(cbuf, acc.at[pl.ds(m * HIST_CP, HIST_CP)])
        return 0
    lax.fori_loop(0, (HIST_NCP + NS - 1) // NS, _zacc, 0)
    plsc.subcore_barrier()

    def _scatter_rows(nrows):
        def go(_=None):
            for q in range(nrows):
                pltpu.sync_copy(ones_v, acc.at[dstv.at[q]], add=True)
        return go

    def _chunk(i, _):
        k = w + i * (NC * NS)

        @pl.when(k < N_FULL_CHUNKS)
        def _():
            pltpu.sync_copy(dst_hbm.at[pl.ds(k * CHUNK_ROWS, CHUNK_ROWS)], dstv)
            _scatter_rows(CHUNK_ROWS)()
        return 0
    lax.fori_loop(0, (N_FULL_CHUNKS + NC * NS - 1) // (NC * NS), _chunk, 0)

    @pl.when(w == HIST_REM_W)
    def _():
        pltpu.sync_copy(
            dst_hbm.at[pl.ds(N_FULL_CHUNKS * CHUNK_ROWS, REM_ROWS)],
            dstv.at[pl.ds(0, REM_ROWS)])
        _scatter_rows(REM_ROWS)()

    plsc.subcore_barrier()

    def _copy_out(i, _):
        m = s + i * NS

        @pl.when(m < HIST_NCP)
        def _():
            off = m * HIST_CP
            pltpu.sync_copy(acc.at[pl.ds(off, HIST_CP)], cbuf)

            @pl.when(c == 0)
            def _():
                pltpu.sync_copy(cbuf, deg0_hbm.at[pl.ds(off, HIST_CP)])

            @pl.when(c == 1)
            def _():
                pltpu.sync_copy(cbuf, deg1_hbm.at[pl.ds(off, HIST_CP)])
        return 0
    lax.fori_loop(0, (HIST_NCP + NS - 1) // NS, _copy_out, 0)


@functools.lru_cache(maxsize=1)
def _hist_kernel():
    return pl.kernel(
        _hist_body,
        out_type=(jax.ShapeDtypeStruct((N_FLAT,), jnp.float32),
                  jax.ShapeDtypeStruct((N_FLAT,), jnp.float32)),
        mesh=_get_mesh(),
        scratch_types=dict(
            acc=pltpu.VMEM_SHARED((N_FLAT,), jnp.float32),
            dstv=pltpu.VMEM((CHUNK_ROWS, 128), jnp.int32),
            ones_v=pltpu.VMEM((128,), jnp.float32),
            cbuf=pltpu.VMEM((HIST_CP,), jnp.float32),
        ),
        compiler_params=pltpu.CompilerParams(use_tc_tiling_on_sc=False),
    )


def _mm_body(x_ref, w_ref, deg0_ref, deg1_ref, hh_ref, dinv_ref):
    dsum = deg0_ref[...] + deg1_ref[...] + 1.0
    dinv = lax.rsqrt(dsum)
    h = jnp.dot(x_ref[...], w_ref[...], preferred_element_type=jnp.float32)
    hh_ref[...] = h * dinv
    dinv_ref[...] = dinv


def _run_mm(x_in, gcn_W, deg0, deg1):
    R = 3200
    grid = (N_FLAT // R,)
    return pl.pallas_call(
        _mm_body,
        grid=grid,
        in_specs=[
            pl.BlockSpec((R, N_FEAT), lambda i: (i, 0)),
            pl.BlockSpec((N_FEAT, HIDDEN), lambda i: (0, 0)),
            pl.BlockSpec((R, 1), lambda i: (i, 0)),
            pl.BlockSpec((R, 1), lambda i: (i, 0)),
        ],
        out_specs=[
            pl.BlockSpec((R, HIDDEN), lambda i: (i, 0)),
            pl.BlockSpec((R, 1), lambda i: (i, 0)),
        ],
        out_shape=[
            jax.ShapeDtypeStruct((N_FLAT, HIDDEN), jnp.float32),
            jax.ShapeDtypeStruct((N_FLAT, 1), jnp.float32),
        ],
    )(x_in, gcn_W, deg0.reshape(N_FLAT, 1), deg1.reshape(N_FLAT, 1))


def _edge_body(src_hbm, dst_hbm, hh_hbm, seg_hbm,
               acc, srcv, dstv, sel_src, sel_rel, rel_stage, rows,
               zbuf, cbuf, semL, semG, semS):
    c = lax.axis_index("c")
    s = lax.axis_index("s")

    zeros16 = jnp.zeros((16,), jnp.float32)
    zeros16i = jnp.zeros((16,), jnp.int32)
    dump16 = jnp.full((16,), DUMP, jnp.int32)
    lane16 = jax.lax.broadcasted_iota(jnp.int32, (16,), 0)

    def _zz(i, _):
        r = i // 2
        col = (i % 2) * 16
        zbuf[r, pl.ds(col, 16)] = zeros16
        return 0
    lax.fori_loop(0, CP_ROWS * 2, _zz, 0)

    def _issue_loads(k):
        off = k * CHUNK_ROWS
        pltpu.async_copy(src_hbm.at[pl.ds(off, CHUNK_ROWS)], srcv, semL)
        pltpu.async_copy(dst_hbm.at[pl.ds(off, CHUNK_ROWS)], dstv, semL)

    def _wait_loads():
        pltpu.make_async_copy(src_hbm.at[pl.ds(0, CHUNK_ROWS)], srcv, semL).wait()
        pltpu.make_async_copy(dst_hbm.at[pl.ds(0, CHUNK_ROWS)], dstv, semL).wait()

    def _compress(nrows, base, cnt):
        def _cv(jj, cnt):
            for u in range(4):
                j = jj * 4 + u
                q = j // 8
                col = (j % 8) * 16
                d = dstv[q, pl.ds(col, 16)]
                r = d - base
                m = (r >= 0) & (r < RANGE)
                mi = jnp.where(m, 1, 0).astype(jnp.int32)
                pf = plsc.cumsum(mi)
                pos = jnp.where(m, cnt + pf - 1, 1535)
                plsc.store_scatter(sel_src, [pos], srcv[q, pl.ds(col, 16)])
                plsc.store_scatter(sel_rel, [pos], jnp.where(m, r, DUMP))
                cnt = cnt + pf[15]
            return cnt
        cnt = lax.fori_loop(0, nrows * 2, _cv, cnt)
        pad = (-cnt) & 7
        sel_src[pl.ds(cnt, 16)] = zeros16i
        sel_rel[pl.ds(cnt, 16)] = dump16
        return cnt + pad

    def _drain_scatters(_):
        for q in range(4):
            pltpu.make_async_copy(rows.at[pl.ds(q * 128, 128)],
                                  acc.at[rel_stage.at[q]], semS).wait()
        return 0

    def _flush512(off0, pending):
        off0 = pl.multiple_of(off0, 8)
        lax.cond(pending == 1, _drain_scatters, lambda _: 0, 0)
        gds = [pltpu.async_copy(
                   hh_hbm.at[sel_src.at[pl.ds(off0 + q * 128, 128)]],
                   rows.at[pl.ds(q * 128, 128)], semG)
               for q in range(4)]
        def _mv(j, _):
            q = j // 8
            col = (j % 8) * 16
            rel_stage[q, pl.ds(col, 16)] = sel_rel[pl.ds(off0 + q * 128 + col, 16)]
            return 0
        lax.fori_loop(0, 32, _mv, 0)
        for gd in gds:
            gd.wait()
        for q in range(4):
            pltpu.async_copy(rows.at[pl.ds(q * 128, 128)],
                             acc.at[rel_stage.at[q]], semS, add=True)
        return 1

    def _maybe_flush(cnt, pending):
        def fl(args):
            cnt, pending = args
            pending = _flush512(cnt - 512, pending)
            return cnt - 512, pending
        return lax.cond(cnt >= 512, fl, lambda a: a, (cnt, pending))

    for p in range(N_PASSES):
        base = (c * N_PASSES + p) * RANGE

        def _zacc(i, _):
            m = s + i * NS

            @pl.when(m < N_CP)
            def _():
                pltpu.sync_copy(zbuf, acc.at[pl.ds(m * CP_ROWS, CP_ROWS)])
            return 0
        lax.fori_loop(0, (N_CP + NS - 1) // NS, _zacc, 0)
        plsc.subcore_barrier()

        _issue_loads(s)

        def _chunk(i, st):
            k = s + i * NS

            def do(st):
                cnt, pending = st
                _wait_loads()
                cnt = _compress(CHUNK_ROWS, base, cnt)

                @pl.when(k + NS < N_FULL_CHUNKS)
                def _():
                    _issue_loads(k + NS)
                cnt, pending = _maybe_flush(cnt, pending)
                cnt, pending = _maybe_flush(cnt, pending)
                return cnt, pending
            return lax.cond(k < N_FULL_CHUNKS, do, lambda st: st, st)
        cnt, pending = lax.fori_loop(
            0, (N_FULL_CHUNKS + NS - 1) // NS, _chunk, (0, 0))

        def _rem(st):
            cnt, pending = st
            pltpu.sync_copy(src_hbm.at[pl.ds(N_FULL_CHUNKS * CHUNK_ROWS, REM_ROWS)],
                            srcv.at[pl.ds(0, REM_ROWS)])
            pltpu.sync_copy(dst_hbm.at[pl.ds(N_FULL_CHUNKS * CHUNK_ROWS, REM_ROWS)],
                            dstv.at[pl.ds(0, REM_ROWS)])
            cnt = _compress(REM_ROWS, base, cnt)
            return _maybe_flush(cnt, pending)
        cnt, pending = lax.cond(s == EDGE_REM_S, _rem, lambda st: st,
                                (cnt, pending))

        def _fin(st):
            cnt, pending = st
            def _padloop(i, _):
                off = cnt + i * 16

                @pl.when(off < 512)
                def _():
                    sel_src[pl.ds(off, 16)] = zeros16i
                    sel_rel[pl.ds(off, 16)] = dump16
                return 0
            lax.fori_loop(0, 32, _padloop, 0)
            return cnt, _flush512(0, pending)
        cnt, pending = lax.cond(cnt > 0, _fin, lambda st: st, (cnt, pending))

        lax.cond(pending == 1, _drain_scatters, lambda _: 0, 0)

        plsc.subcore_barrier()

        def _cacc(i, _):
            m = s + i * NS

            @pl.when(m < N_CP)
            def _():
                row = m * CP_ROWS
                pltpu.sync_copy(acc.at[pl.ds(row, CP_ROWS)], cbuf)
                pltpu.sync_copy(cbuf, seg_hbm.at[pl.ds(base + row, CP_ROWS)])
            return 0
        lax.fori_loop(0, (N_CP + NS - 1) // NS, _cacc, 0)


@functools.lru_cache(maxsize=1)
def _edge_kernel():
    return pl.kernel(
        _edge_body,
        out_type=jax.ShapeDtypeStruct((N_FLAT, HIDDEN), jnp.float32),
        mesh=_get_mesh(),
        scratch_types=dict(
        acc=pltpu.VMEM_SHARED((RANGE + 1, HIDDEN), jnp.float32),
        srcv=pltpu.VMEM((CHUNK_ROWS, 128), jnp.int32),
        dstv=pltpu.VMEM((CHUNK_ROWS, 128), jnp.int32),
        sel_src=pltpu.VMEM((1536,), jnp.int32),
        sel_rel=pltpu.VMEM((1536,), jnp.int32),
        rel_stage=pltpu.VMEM((4, 128), jnp.int32),
        rows=pltpu.VMEM((512, HIDDEN), jnp.float32),
            zbuf=pltpu.VMEM((CP_ROWS, HIDDEN), jnp.float32),
            cbuf=pltpu.VMEM((CP_ROWS, HIDDEN), jnp.float32),
            semL=pltpu.SemaphoreType.DMA,
            semG=pltpu.SemaphoreType.DMA,
            semS=pltpu.SemaphoreType.DMA,
        ),
        compiler_params=pltpu.CompilerParams(
            use_tc_tiling_on_sc=False, needs_layout_passes=False),
    )


def _lstm_body(hh_ref, seg_ref, dinv_ref, gcn_b_ref, wcat_ref,
               b4_ref, fcw_ref, fcb_ref, out_ref):
    B = hh_ref.shape[0]
    gb = gcn_b_ref[...]
    wcat = wcat_ref[...]
    b4 = b4_ref[...]
    hs = jnp.zeros((B, HIDDEN), jnp.float32)
    cs = jnp.zeros((B, HIDDEN), jnp.float32)
    for t in range(SEQ_LEN):
        xt = dinv_ref[:, t, :] * (seg_ref[:, t, :] + hh_ref[:, t, :]) + gb
        xt = jax.nn.relu(xt)
        xh = jnp.concatenate([xt, hs], axis=1)
        gates = jnp.dot(xh, wcat, preferred_element_type=jnp.float32) + b4
        gi = jax.nn.sigmoid(gates[:, 0 * HIDDEN:1 * HIDDEN])
        gf = jax.nn.sigmoid(gates[:, 1 * HIDDEN:2 * HIDDEN])
        gg = jnp.tanh(gates[:, 2 * HIDDEN:3 * HIDDEN])
        go = jax.nn.sigmoid(gates[:, 3 * HIDDEN:4 * HIDDEN])
        cs = gf * cs + gi * gg
        hs = go * jnp.tanh(cs)
    out_ref[...] = (jnp.dot(hs, fcw_ref[...], preferred_element_type=jnp.float32)
                    + fcb_ref[...])


def _run_lstm(hh3, seg3, dinv3, gcn_b2, wcat, b4, fcwT, fcb2):
    B = 1000
    grid = (N_NODES // B,)
    return pl.pallas_call(
        _lstm_body,
        grid=grid,
        in_specs=[
            pl.BlockSpec((B, SEQ_LEN, HIDDEN), lambda i: (i, 0, 0)),
            pl.BlockSpec((B, SEQ_LEN, HIDDEN), lambda i: (i, 0, 0)),
            pl.BlockSpec((B, SEQ_LEN, 1), lambda i: (i, 0, 0)),
            pl.BlockSpec((1, HIDDEN), lambda i: (0, 0)),
            pl.BlockSpec((2 * HIDDEN, 4 * HIDDEN), lambda i: (0, 0)),
            pl.BlockSpec((1, 4 * HIDDEN), lambda i: (0, 0)),
            pl.BlockSpec((HIDDEN, 1), lambda i: (0, 0)),
            pl.BlockSpec((1, 1), lambda i: (0, 0)),
        ],
        out_specs=pl.BlockSpec((B, 1), lambda i: (i, 0)),
        out_shape=jax.ShapeDtypeStruct((N_NODES, 1), jnp.float32),
    )(hh3, seg3, dinv3, gcn_b2, wcat, b4, fcwT, fcb2)


def kernel(x, edge_index, gcn_W, gcn_b, W_ih, W_hh, b_ih, b_hh, fc_W, fc_b):
    x_in = x.reshape(N_FLAT, N_FEAT)
    src2 = edge_index[0].reshape(EROWS, 128)
    dst2 = edge_index[1].reshape(EROWS, 128)

    deg0, deg1 = _hist_kernel()(dst2)
    hh, dinv = _run_mm(x_in, gcn_W, deg0, deg1)
    segsum = _edge_kernel()(src2, dst2, hh)

    out = _run_lstm(
        hh.reshape(N_NODES, SEQ_LEN, HIDDEN),
        segsum.reshape(N_NODES, SEQ_LEN, HIDDEN),
        dinv.reshape(N_NODES, SEQ_LEN, 1),
        gcn_b.reshape(1, HIDDEN),
        jnp.concatenate([W_ih.T, W_hh.T], axis=0),
        (b_ih + b_hh).reshape(1, 4 * HIDDEN),
        fc_W.T,
        fc_b.reshape(1, 1),
    )
    return out


# speedup vs baseline: 1.1540x; 1.1284x over previous
import functools

import jax
import jax.numpy as jnp
from jax import lax
from jax.experimental import pallas as pl
from jax.experimental.pallas import tpu as pltpu
from jax.experimental.pallas import tpu_sc as plsc

N_NODES = 50000
SEQ_LEN = 8
N_FEAT = 128
HIDDEN = 32
N_EDGES = 624000
N_FLAT = N_NODES * SEQ_LEN

NC = 2
NS = 16

EROWS = N_EDGES // 128
CHUNK_ROWS = 8
N_FULL_CHUNKS = EROWS // CHUNK_ROWS
REM_ROWS = EROWS - N_FULL_CHUNKS * CHUNK_ROWS
HIST_REM_W = N_FULL_CHUNKS % (2 * 16)
EDGE_REM_S = N_FULL_CHUNKS % 16

N_PASSES = 4
RANGE = N_FLAT // (NC * N_PASSES)
DUMP = RANGE
CP_ROWS = 80
N_CP = RANGE // CP_ROWS
HIST_CP = 4000
HIST_NCP = N_FLAT // HIST_CP
SEL_CAP = 1664

@functools.lru_cache(maxsize=1)
def _get_mesh():
    return plsc.VectorSubcoreMesh(
        core_axis_name="c", subcore_axis_name="s",
        num_cores=NC, num_subcores=NS)


def _hist_body(dst_hbm, deg0_hbm, deg1_hbm, acc, dstv, ones_v, cbuf):
    c = lax.axis_index("c")
    s = lax.axis_index("s")
    w = c * NS + s

    zeros16 = jnp.zeros((16,), jnp.float32)
    ones16 = jnp.ones((16,), jnp.float32)

    def _fill(i, _):
        ones_v[pl.ds(i * 16, 16)] = ones16
        return 0
    lax.fori_loop(0, 8, _fill, 0)

    def _zcb(i, _):
        cbuf[pl.ds(i * 16, 16)] = zeros16
        return 0
    lax.fori_loop(0, HIST_CP // 16, _zcb, 0)

    def _zacc(i, _):
        m = s + i * NS

        @pl.when(m < HIST_NCP)
        def _():
            pltpu.sync_copy(cbuf, acc.at[pl.ds(m * HIST_CP, HIST_CP)])
        return 0
    lax.fori_loop(0, (HIST_NCP + NS - 1) // NS, _zacc, 0)
    plsc.subcore_barrier()

    def _scatter_rows(nrows):
        def go(_=None):
            for q in range(nrows):
                pltpu.sync_copy(ones_v, acc.at[dstv.at[q]], add=True)
        return go

    def _chunk(i, _):
        k = w + i * (NC * NS)

        @pl.when(k < N_FULL_CHUNKS)
        def _():
            pltpu.sync_copy(dst_hbm.at[pl.ds(k * CHUNK_ROWS, CHUNK_ROWS)], dstv)
            _scatter_rows(CHUNK_ROWS)()
        return 0
    lax.fori_loop(0, (N_FULL_CHUNKS + NC * NS - 1) // (NC * NS), _chunk, 0)

    @pl.when(w == HIST_REM_W)
    def _():
        pltpu.sync_copy(
            dst_hbm.at[pl.ds(N_FULL_CHUNKS * CHUNK_ROWS, REM_ROWS)],
            dstv.at[pl.ds(0, REM_ROWS)])
        _scatter_rows(REM_ROWS)()

    plsc.subcore_barrier()

    def _copy_out(i, _):
        m = s + i * NS

        @pl.when(m < HIST_NCP)
        def _():
            off = m * HIST_CP
            pltpu.sync_copy(acc.at[pl.ds(off, HIST_CP)], cbuf)

            @pl.when(c == 0)
            def _():
                pltpu.sync_copy(cbuf, deg0_hbm.at[pl.ds(off, HIST_CP)])

            @pl.when(c == 1)
            def _():
                pltpu.sync_copy(cbuf, deg1_hbm.at[pl.ds(off, HIST_CP)])
        return 0
    lax.fori_loop(0, (HIST_NCP + NS - 1) // NS, _copy_out, 0)


@functools.lru_cache(maxsize=1)
def _hist_kernel():
    return pl.kernel(
        _hist_body,
        out_type=(jax.ShapeDtypeStruct((N_FLAT,), jnp.float32),
                  jax.ShapeDtypeStruct((N_FLAT,), jnp.float32)),
        mesh=_get_mesh(),
        scratch_types=dict(
            acc=pltpu.VMEM_SHARED((N_FLAT,), jnp.float32),
            dstv=pltpu.VMEM((CHUNK_ROWS, 128), jnp.int32),
            ones_v=pltpu.VMEM((128,), jnp.float32),
            cbuf=pltpu.VMEM((HIST_CP,), jnp.float32),
        ),
        compiler_params=pltpu.CompilerParams(use_tc_tiling_on_sc=False),
    )


def _mm_body(x_ref, w_ref, deg0_ref, deg1_ref, hh_ref, dinv_ref):
    dsum = deg0_ref[...] + deg1_ref[...] + 1.0
    dinv = lax.rsqrt(dsum)
    h = jnp.dot(x_ref[...], w_ref[...], preferred_element_type=jnp.float32)
    hh_ref[...] = h * dinv
    dinv_ref[...] = dinv


def _run_mm(x_in, gcn_W, deg0, deg1):
    R = 3200
    grid = (N_FLAT // R,)
    return pl.pallas_call(
        _mm_body,
        grid=grid,
        in_specs=[
            pl.BlockSpec((R, N_FEAT), lambda i: (i, 0)),
            pl.BlockSpec((N_FEAT, HIDDEN), lambda i: (0, 0)),
            pl.BlockSpec((R, 1), lambda i: (i, 0)),
            pl.BlockSpec((R, 1), lambda i: (i, 0)),
        ],
        out_specs=[
            pl.BlockSpec((R, HIDDEN), lambda i: (i, 0)),
            pl.BlockSpec((R, 1), lambda i: (i, 0)),
        ],
        out_shape=[
            jax.ShapeDtypeStruct((N_FLAT, HIDDEN), jnp.float32),
            jax.ShapeDtypeStruct((N_FLAT, 1), jnp.float32),
        ],
    )(x_in, gcn_W, deg0.reshape(N_FLAT, 1), deg1.reshape(N_FLAT, 1))


def _edge_body(src_hbm, dst_hbm, hh_hbm, seg_hbm,
               acc, srcv, dstv, sel_src, sel_rel, rel_stage, rows,
               zbuf, cbuf, semL, semG, semS):
    c = lax.axis_index("c")
    s = lax.axis_index("s")

    zeros16 = jnp.zeros((16,), jnp.float32)
    zeros16i = jnp.zeros((16,), jnp.int32)
    dump16 = jnp.full((16,), DUMP, jnp.int32)
    lane16 = jax.lax.broadcasted_iota(jnp.int32, (16,), 0)

    def _zz(i, _):
        r = i // 2
        col = (i % 2) * 16
        zbuf[r, pl.ds(col, 16)] = zeros16
        return 0
    lax.fori_loop(0, CP_ROWS * 2, _zz, 0)

    def _issue_loads(k):
        off = k * CHUNK_ROWS
        pltpu.async_copy(src_hbm.at[pl.ds(off, CHUNK_ROWS)], srcv, semL)
        pltpu.async_copy(dst_hbm.at[pl.ds(off, CHUNK_ROWS)], dstv, semL)

    def _wait_loads():
        pltpu.make_async_copy(src_hbm.at[pl.ds(0, CHUNK_ROWS)], srcv, semL).wait()
        pltpu.make_async_copy(dst_hbm.at[pl.ds(0, CHUNK_ROWS)], dstv, semL).wait()

    def _compress(nrows, base, cnt):
        def _cv(jj, cnt):
            for u in range(4):
                j = jj * 4 + u
                q = j // 8
                col = (j % 8) * 16
                d = dstv[q, pl.ds(col, 16)]
                r = d - base
                m = (r >= 0) & (r < RANGE)
                mi = jnp.where(m, 1, 0).astype(jnp.int32)
                pf = plsc.cumsum(mi)
                pos = jnp.where(m, cnt + pf - 1, SEL_CAP - 1)
                plsc.store_scatter(sel_src, [pos], srcv[q, pl.ds(col, 16)])
                plsc.store_scatter(sel_rel, [pos], jnp.where(m, r, DUMP))
                cnt = cnt + pf[15]
            return cnt
        cnt = lax.fori_loop(0, nrows * 2, _cv, cnt)
        pad = (-cnt) & 7
        sel_src[pl.ds(cnt, 16)] = zeros16i
        sel_rel[pl.ds(cnt, 16)] = dump16
        return cnt + pad

    def _drain_scatters(_):
        for q in range(4):
            pltpu.make_async_copy(rows.at[pl.ds(q * 128, 128)],
                                  acc.at[rel_stage.at[q]], semS).wait()
        return 0

    def _flush512(off0, pending):
        off0 = pl.multiple_of(off0, 8)
        lax.cond(pending == 1, _drain_scatters, lambda _: 0, 0)
        gds = [pltpu.async_copy(
                   hh_hbm.at[sel_src.at[pl.ds(off0 + q * 128, 128)]],
                   rows.at[pl.ds(q * 128, 128)], semG)
               for q in range(4)]
        def _mv(j, _):
            q = j // 8
            col = (j % 8) * 16
            rel_stage[q, pl.ds(col, 16)] = sel_rel[pl.ds(off0 + q * 128 + col, 16)]
            return 0
        lax.fori_loop(0, 32, _mv, 0)
        for gd in gds:
            gd.wait()
        for q in range(4):
            pltpu.async_copy(rows.at[pl.ds(q * 128, 128)],
                             acc.at[rel_stage.at[q]], semS, add=True)
        return 1

    def _maybe_flush(cnt, pending):
        def fl(args):
            cnt, pending = args
            pending = _flush512(cnt - 512, pending)
            return cnt - 512, pending
        return lax.cond(cnt >= 512, fl, lambda a: a, (cnt, pending))

    for p in range(N_PASSES):
        base = (c * N_PASSES + p) * RANGE

        def _zacc(i, _):
            m = s + i * NS

            @pl.when(m < N_CP)
            def _():
                pltpu.sync_copy(zbuf, acc.at[pl.ds(m * CP_ROWS, CP_ROWS)])
            return 0
        lax.fori_loop(0, (N_CP + NS - 1) // NS, _zacc, 0)
        plsc.subcore_barrier()

        _issue_loads(s)

        def _chunk(i, st):
            k = s + i * NS

            def do(st):
                cnt, pending = st
                _wait_loads()
                cnt = _compress(CHUNK_ROWS, base, cnt)

                @pl.when(k + NS < N_FULL_CHUNKS)
                def _():
                    _issue_loads(k + NS)
                cnt, pending = _maybe_flush(cnt, pending)
                cnt, pending = _maybe_flush(cnt, pending)
                cnt, pending = _maybe_flush(cnt, pending)
                return cnt, pending
            return lax.cond(k < N_FULL_CHUNKS, do, lambda st: st, st)
        cnt, pending = lax.fori_loop(
            0, (N_FULL_CHUNKS + NS - 1) // NS, _chunk, (0, 0))

        def _rem(st):
            cnt, pending = st
            pltpu.sync_copy(src_hbm.at[pl.ds(N_FULL_CHUNKS * CHUNK_ROWS, REM_ROWS)],
                            srcv.at[pl.ds(0, REM_ROWS)])
            pltpu.sync_copy(dst_hbm.at[pl.ds(N_FULL_CHUNKS * CHUNK_ROWS, REM_ROWS)],
                            dstv.at[pl.ds(0, REM_ROWS)])
            cnt = _compress(REM_ROWS, base, cnt)
            return _maybe_flush(cnt, pending)
        cnt, pending = lax.cond(s == EDGE_REM_S, _rem, lambda st: st,
                                (cnt, pending))

        def _fin(st):
            cnt, pending = st
            def _padloop(i, _):
                off = cnt + i * 16

                @pl.when(off < 512)
                def _():
                    sel_src[pl.ds(off, 16)] = zeros16i
                    sel_rel[pl.ds(off, 16)] = dump16
                return 0
            lax.fori_loop(0, 32, _padloop, 0)
            return cnt, _flush512(0, pending)
        cnt, pending = lax.cond(cnt > 0, _fin, lambda st: st, (cnt, pending))

        lax.cond(pending == 1, _drain_scatters, lambda _: 0, 0)

        plsc.subcore_barrier()

        def _cacc(i, _):
            m = s + i * NS

            @pl.when(m < N_CP)
            def _():
                row = m * CP_ROWS
                pltpu.sync_copy(acc.at[pl.ds(row, CP_ROWS)], cbuf)
                pltpu.sync_copy(cbuf, seg_hbm.at[pl.ds(base + row, CP_ROWS)])
            return 0
        lax.fori_loop(0, (N_CP + NS - 1) // NS, _cacc, 0)


@functools.lru_cache(maxsize=1)
def _edge_kernel():
    return pl.kernel(
        _edge_body,
        out_type=jax.ShapeDtypeStruct((N_FLAT, HIDDEN), jnp.float32),
        mesh=_get_mesh(),
        scratch_types=dict(
        acc=pltpu.VMEM_SHARED((RANGE + 1, HIDDEN), jnp.float32),
        srcv=pltpu.VMEM((CHUNK_ROWS, 128), jnp.int32),
        dstv=pltpu.VMEM((CHUNK_ROWS, 128), jnp.int32),
        sel_src=pltpu.VMEM((SEL_CAP,), jnp.int32),
        sel_rel=pltpu.VMEM((SEL_CAP,), jnp.int32),
        rel_stage=pltpu.VMEM((4, 128), jnp.int32),
        rows=pltpu.VMEM((512, HIDDEN), jnp.float32),
            zbuf=pltpu.VMEM((CP_ROWS, HIDDEN), jnp.float32),
            cbuf=pltpu.VMEM((CP_ROWS, HIDDEN), jnp.float32),
            semL=pltpu.SemaphoreType.DMA,
            semG=pltpu.SemaphoreType.DMA,
            semS=pltpu.SemaphoreType.DMA,
        ),
        compiler_params=pltpu.CompilerParams(
            use_tc_tiling_on_sc=False, needs_layout_passes=False),
    )


def _lstm_body(hh_ref, seg_ref, dinv_ref, gcn_b_ref, wcat_ref,
               b4_ref, fcw_ref, fcb_ref, out_ref):
    B = hh_ref.shape[0]
    gb = gcn_b_ref[...]
    wcat = wcat_ref[...]
    b4 = b4_ref[...]
    hs = jnp.zeros((B, HIDDEN), jnp.float32)
    cs = jnp.zeros((B, HIDDEN), jnp.float32)
    for t in range(SEQ_LEN):
        xt = dinv_ref[:, t, :] * (seg_ref[:, t, :] + hh_ref[:, t, :]) + gb
        xt = jax.nn.relu(xt)
        xh = jnp.concatenate([xt, hs], axis=1)
        gates = jnp.dot(xh, wcat, preferred_element_type=jnp.float32) + b4
        gi = jax.nn.sigmoid(gates[:, 0 * HIDDEN:1 * HIDDEN])
        gf = jax.nn.sigmoid(gates[:, 1 * HIDDEN:2 * HIDDEN])
        gg = jnp.tanh(gates[:, 2 * HIDDEN:3 * HIDDEN])
        go = jax.nn.sigmoid(gates[:, 3 * HIDDEN:4 * HIDDEN])
        cs = gf * cs + gi * gg
        hs = go * jnp.tanh(cs)
    out_ref[...] = (jnp.dot(hs, fcw_ref[...], preferred_element_type=jnp.float32)
                    + fcb_ref[...])


def _run_lstm(hh3, seg3, dinv3, gcn_b2, wcat, b4, fcwT, fcb2):
    B = 1000
    grid = (N_NODES // B,)
    return pl.pallas_call(
        _lstm_body,
        grid=grid,
        in_specs=[
            pl.BlockSpec((B, SEQ_LEN, HIDDEN), lambda i: (i, 0, 0)),
            pl.BlockSpec((B, SEQ_LEN, HIDDEN), lambda i: (i, 0, 0)),
            pl.BlockSpec((B, SEQ_LEN, 1), lambda i: (i, 0, 0)),
            pl.BlockSpec((1, HIDDEN), lambda i: (0, 0)),
            pl.BlockSpec((2 * HIDDEN, 4 * HIDDEN), lambda i: (0, 0)),
            pl.BlockSpec((1, 4 * HIDDEN), lambda i: (0, 0)),
            pl.BlockSpec((HIDDEN, 1), lambda i: (0, 0)),
            pl.BlockSpec((1, 1), lambda i: (0, 0)),
        ],
        out_specs=pl.BlockSpec((B, 1), lambda i: (i, 0)),
        out_shape=jax.ShapeDtypeStruct((N_NODES, 1), jnp.float32),
    )(hh3, seg3, dinv3, gcn_b2, wcat, b4, fcwT, fcb2)


def kernel(x, edge_index, gcn_W, gcn_b, W_ih, W_hh, b_ih, b_hh, fc_W, fc_b):
    x_in = x.reshape(N_FLAT, N_FEAT)
    src2 = edge_index[0].reshape(EROWS, 128)
    dst2 = edge_index[1].reshape(EROWS, 128)

    deg0, deg1 = _hist_kernel()(dst2)
    hh, dinv = _run_mm(x_in, gcn_W, deg0, deg1)
    segsum = _edge_kernel()(src2, dst2, hh)

    out = _run_lstm(
        hh.reshape(N_NODES, SEQ_LEN, HIDDEN),
        segsum.reshape(N_NODES, SEQ_LEN, HIDDEN),
        dinv.reshape(N_NODES, SEQ_LEN, 1),
        gcn_b.reshape(1, HIDDEN),
        jnp.concatenate([W_ih.T, W_hh.T], axis=0),
        (b_ih + b_hh).reshape(1, 4 * HIDDEN),
        fc_W.T,
        fc_b.reshape(1, 1),
    )
    return out
